# speedup vs baseline: 2.7181x; 2.7181x over previous
import math

import jax
import jax.numpy as jnp
from jax.experimental import pallas as pl

_B = 16384
_D = 128
_HALF = 64
_BLK = 2048

_TWO_PI = 2.0 * math.pi


def _fused_body(g_ref, h_ref, age_ref, gt_ref, ht_ref, w_ref, out_ref):
    g = g_ref[...]
    h = h_ref[...]
    age = age_ref[...]
    age = jnp.where(jnp.isnan(age), jnp.zeros_like(age), age)

    gt0 = gt_ref[0:1, :]
    gt1 = gt_ref[1:2, :]
    gt2 = gt_ref[2:3, :]
    ge = jnp.where(g == 0, gt0, jnp.where(g == 1, gt1, gt2))

    ht0 = ht_ref[0:1, :]
    ht1 = ht_ref[1:2, :]
    ht2 = ht_ref[2:3, :]
    he = jnp.where(h == 0, ht0, jnp.where(h == 1, ht1, ht2))

    tab = ge + he

    f = (_TWO_PI * age) * w_ref[...]
    age_emb = jnp.concatenate([jnp.cos(f), jnp.sin(f)], axis=-1)

    out_ref[...] = tab + age_emb


@jax.jit
def kernel(gender_labels, health_labels, age_values, gender_table,
           health_table, fourier_weight):
    g2 = gender_labels.astype(jnp.int32).reshape(_B, 1)
    h2 = health_labels.astype(jnp.int32).reshape(_B, 1)
    a2 = age_values.reshape(_B, 1)
    w = fourier_weight.reshape(1, _HALF)

    grid = (_B // _BLK,)
    return pl.pallas_call(
        _fused_body,
        grid=grid,
        in_specs=[
            pl.BlockSpec((_BLK, 1), lambda i: (i, 0)),
            pl.BlockSpec((_BLK, 1), lambda i: (i, 0)),
            pl.BlockSpec((_BLK, 1), lambda i: (i, 0)),
            pl.BlockSpec((3, _D), lambda i: (0, 0)),
            pl.BlockSpec((3, _D), lambda i: (0, 0)),
            pl.BlockSpec((1, _HALF), lambda i: (0, 0)),
        ],
        out_specs=pl.BlockSpec((_BLK, _D), lambda i: (i, 0)),
        out_shape=jax.ShapeDtypeStruct((_B, _D), jnp.float32),
    )(g2, h2, a2, gender_table, health_table, w)


# speedup vs baseline: 10.1145x; 3.7212x over previous
import math

import numpy as np
import jax
import jax.numpy as jnp
from jax.experimental import pallas as pl

_B = 16384
_D = 128
_HALF = 64
_BLK = 2048
_GROUPS = _BLK // 128

_NK = 8
_COS_C = [(-1.0) ** k * (2.0 * math.pi) ** (2 * k) / math.factorial(2 * k)
          for k in range(_NK)]
_SIN_C = [(-1.0) ** k * (2.0 * math.pi) ** (2 * k + 1) / math.factorial(2 * k + 1)
          for k in range(_NK)]

_lane_is_cos = np.arange(_D) < _HALF
_COEF = np.stack(
    [np.where(_lane_is_cos, c, s).astype(np.float32)
     for c, s in zip(_COS_C, _SIN_C)]
    + [_lane_is_cos.astype(np.float32), (~_lane_is_cos).astype(np.float32)]
)


def _fused_body(g_ref, h_ref, age_ref, gt_ref, ht_ref, w_ref, coef_ref,
                out_ref):
    age = age_ref[...]
    age = jnp.where(jnp.isnan(age), jnp.zeros_like(age), age)
    ageT = jnp.transpose(age)
    gT = jnp.transpose(g_ref[...])
    hT = jnp.transpose(h_ref[...])

    gt0 = gt_ref[0:1, :]
    gt1 = gt_ref[1:2, :]
    gt2 = gt_ref[2:3, :]
    ht0 = ht_ref[0:1, :]
    ht1 = ht_ref[1:2, :]
    ht2 = ht_ref[2:3, :]
    w = w_ref[...]
    mcos = coef_ref[_NK:_NK + 1, :]
    msin = coef_ref[_NK + 1:_NK + 2, :]

    for j in range(_GROUPS):
        a = ageT[:, j:j + 1]
        g = gT[:, j:j + 1]
        h = hT[:, j:j + 1]

        t = a * w
        r = t - jnp.round(t)
        x = r * r
        acc = jnp.broadcast_to(coef_ref[_NK - 1:_NK, :], t.shape)
        for k in range(_NK - 2, -1, -1):
            acc = acc * x + coef_ref[k:k + 1, :]
        trig = acc * (mcos + r * msin)

        ge = jnp.where(g == 0, gt0, jnp.where(g == 1, gt1, gt2))
        he = jnp.where(h == 0, ht0, jnp.where(h == 1, ht1, ht2))

        out_ref[j * 128:(j + 1) * 128, :] = trig + ge + he


@jax.jit
def kernel(gender_labels, health_labels, age_values, gender_table,
           health_table, fourier_weight):
    g2 = gender_labels.astype(jnp.int32).reshape(_B // _D, _D)
    h2 = health_labels.astype(jnp.int32).reshape(_B // _D, _D)
    a2 = age_values.reshape(_B // _D, _D)
    wrow = fourier_weight.reshape(1, _HALF)
    wcat = jnp.concatenate([wrow, wrow], axis=1)
    coef = jnp.asarray(_COEF)

    grid = (_B // _BLK,)
    return pl.pallas_call(
        _fused_body,
        grid=grid,
        in_specs=[
            pl.BlockSpec((_GROUPS, _D), lambda i: (i, 0)),
            pl.BlockSpec((_GROUPS, _D), lambda i: (i, 0)),
            pl.BlockSpec((_GROUPS, _D), lambda i: (i, 0)),
            pl.BlockSpec((3, _D), lambda i: (0, 0)),
            pl.BlockSpec((3, _D), lambda i: (0, 0)),
            pl.BlockSpec((1, _D), lambda i: (0, 0)),
            pl.BlockSpec((_NK + 2, _D), lambda i: (0, 0)),
        ],
        out_specs=pl.BlockSpec((_BLK, _D), lambda i: (i, 0)),
        out_shape=jax.ShapeDtypeStruct((_B, _D), jnp.float32),
    )(g2, h2, a2, gender_table, health_table, wcat, coef)


# speedup vs baseline: 12.4320x; 1.2291x over previous
import jax
import jax.numpy as jnp
import numpy as np
from jax import lax
from jax.experimental import pallas as pl

_B = 16384
_D = 128
_HALF = 64
_BLK = 2048
_NBLK = _B // _BLK

_CC = [0.9999166471955647, -19.729117422607995, 64.65317568647825,
       -82.33816398041094, 45.607153997855676]
_SC = [6.2831361411988205, -41.33575801971555, 81.43700404386047,
       -74.87892931605455, 33.54146240741102]
_NK = 5

_lane_is_cos = np.arange(_D) < _HALF
_COEF = np.stack(
    [np.where(_lane_is_cos, c, s).astype(np.float32)
     for c, s in zip(_CC, _SC)]
    + [_lane_is_cos.astype(np.float32), (~_lane_is_cos).astype(np.float32)]
)


def _fused_body(g_ref, h_ref, age_ref, w3_ref, t16_ref, coef_ref, out_ref):
    age = age_ref[0]
    age = jnp.where(jnp.isnan(age), jnp.zeros_like(age), age)
    a_hi = age.astype(jnp.bfloat16)
    a_lo = (age - a_hi.astype(jnp.float32)).astype(jnp.bfloat16)
    lhs_t = jnp.concatenate([a_hi, a_lo, a_hi], axis=0)

    g = g_ref[0]
    h = h_ref[0]
    iot = lax.broadcasted_iota(jnp.int32, (8, _BLK), 0)
    oh8 = ((iot == g) | (iot == (h + 3))).astype(jnp.bfloat16)
    lhs_tab = jnp.concatenate([oh8, oh8], axis=0)

    dn = (((0,), (0,)), ((), ()))
    t = lax.dot_general(lhs_t, w3_ref[...], dn,
                        preferred_element_type=jnp.float32)
    tab = lax.dot_general(lhs_tab, t16_ref[...], dn,
                          preferred_element_type=jnp.float32)

    r = t - jnp.round(t)
    x = r * r
    acc = jnp.broadcast_to(coef_ref[_NK - 1:_NK, :], t.shape)
    for k in range(_NK - 2, -1, -1):
        acc = acc * x + coef_ref[k:k + 1, :]
    m = coef_ref[_NK:_NK + 1, :] + r * coef_ref[_NK + 1:_NK + 2, :]
    out_ref[...] = acc * m + tab


@jax.jit
def kernel(gender_labels, health_labels, age_values, gender_table,
           health_table, fourier_weight):
    g3 = gender_labels.astype(jnp.int32).reshape(_NBLK, 1, _BLK)
    h3 = health_labels.astype(jnp.int32).reshape(_NBLK, 1, _BLK)
    a3 = age_values.reshape(_NBLK, 1, _BLK)

    wrow = fourier_weight.reshape(1, _HALF)
    wcat = jnp.concatenate([wrow, wrow], axis=1)
    w_hi = wcat.astype(jnp.bfloat16)
    w_lo = (wcat - w_hi.astype(jnp.float32)).astype(jnp.bfloat16)
    w3 = jnp.concatenate([w_hi, w_hi, w_lo], axis=0)

    gt_hi = gender_table.astype(jnp.bfloat16)
    gt_lo = (gender_table - gt_hi.astype(jnp.float32)).astype(jnp.bfloat16)
    ht_hi = health_table.astype(jnp.bfloat16)
    ht_lo = (health_table - ht_hi.astype(jnp.float32)).astype(jnp.bfloat16)
    z2 = jnp.zeros((2, _D), jnp.bfloat16)
    t16 = jnp.concatenate([gt_hi, ht_hi, z2, gt_lo, ht_lo, z2], axis=0)

    coef = jnp.asarray(_COEF)

    grid = (_NBLK,)
    return pl.pallas_call(
        _fused_body,
        grid=grid,
        in_specs=[
            pl.BlockSpec((1, 1, _BLK), lambda i: (i, 0, 0)),
            pl.BlockSpec((1, 1, _BLK), lambda i: (i, 0, 0)),
            pl.BlockSpec((1, 1, _BLK), lambda i: (i, 0, 0)),
            pl.BlockSpec((3, _D), lambda i: (0, 0)),
            pl.BlockSpec((16, _D), lambda i: (0, 0)),
            pl.BlockSpec((_NK + 2, _D), lambda i: (0, 0)),
        ],
        out_specs=pl.BlockSpec((_BLK, _D), lambda i: (i, 0)),
        out_shape=jax.ShapeDtypeStruct((_B, _D), jnp.float32),
    )(g3, h3, a3, w3, t16, coef)


# speedup vs baseline: 13.0719x; 1.0515x over previous
import jax
import jax.numpy as jnp
import numpy as np
from jax import lax
from jax.experimental import pallas as pl

_B = 16384
_D = 128
_HALF = 64
_BLK = 2048
_NBLK = _B // _BLK
_GROUPS = _BLK // 128

_CC = [0.9999166471955647, -19.729117422607995, 64.65317568647825,
       -82.33816398041094, 45.607153997855676]
_SC = [6.2831361411988205, -41.33575801971555, 81.43700404386047,
       -74.87892931605455, 33.54146240741102]
_NK = 5

_lane_is_cos = np.arange(_D) < _HALF
_COEF = np.stack(
    [np.where(_lane_is_cos, c, s).astype(np.float32)
     for c, s in zip(_CC, _SC)]
    + [_lane_is_cos.astype(np.float32), (~_lane_is_cos).astype(np.float32)]
)


def _fused_body(g_ref, h_ref, age_ref, w_ref, t16_ref, coef_ref, out_ref):
    age = age_ref[...]
    age = jnp.where(jnp.isnan(age), jnp.zeros_like(age), age)
    ageT = jnp.transpose(age)

    g = g_ref[0]
    h = h_ref[0]
    iot = lax.broadcasted_iota(jnp.int32, (8, _BLK), 0)
    oh8 = ((iot == g) | (iot == (h + 3))).astype(jnp.bfloat16)
    lhs_tab = jnp.concatenate([oh8, oh8], axis=0)
    dn = (((0,), (0,)), ((), ()))
    tab = lax.dot_general(lhs_tab, t16_ref[...], dn,
                          preferred_element_type=jnp.float32)

    w = w_ref[...]
    mcos = coef_ref[_NK:_NK + 1, :]
    msin = coef_ref[_NK + 1:_NK + 2, :]

    for j in range(_GROUPS):
        a = ageT[:, j:j + 1]
        t = a * w
        r = t - jnp.round(t)
        x = r * r
        acc = jnp.broadcast_to(coef_ref[_NK - 1:_NK, :], t.shape)
        for k in range(_NK - 2, -1, -1):
            acc = acc * x + coef_ref[k:k + 1, :]
        m = mcos + r * msin
        out_ref[j * 128:(j + 1) * 128, :] = (
            acc * m + tab[j * 128:(j + 1) * 128, :])


@jax.jit
def kernel(gender_labels, health_labels, age_values, gender_table,
           health_table, fourier_weight):
    g3 = gender_labels.astype(jnp.int32).reshape(_NBLK, 1, _BLK)
    h3 = health_labels.astype(jnp.int32).reshape(_NBLK, 1, _BLK)
    a2 = age_values.reshape(_B // _D, _D)

    wrow = fourier_weight.reshape(1, _HALF)
    wcat = jnp.concatenate([wrow, wrow], axis=1)

    gt_hi = gender_table.astype(jnp.bfloat16)
    gt_lo = (gender_table - gt_hi.astype(jnp.float32)).astype(jnp.bfloat16)
    ht_hi = health_table.astype(jnp.bfloat16)
    ht_lo = (health_table - ht_hi.astype(jnp.float32)).astype(jnp.bfloat16)
    z2 = jnp.zeros((2, _D), jnp.bfloat16)
    t16 = jnp.concatenate([gt_hi, ht_hi, z2, gt_lo, ht_lo, z2], axis=0)

    coef = jnp.asarray(_COEF)

    grid = (_NBLK,)
    return pl.pallas_call(
        _fused_body,
        grid=grid,
        in_specs=[
            pl.BlockSpec((1, 1, _BLK), lambda i: (i, 0, 0)),
            pl.BlockSpec((1, 1, _BLK), lambda i: (i, 0, 0)),
            pl.BlockSpec((_GROUPS, _D), lambda i: (i, 0)),
            pl.BlockSpec((1, _D), lambda i: (0, 0)),
            pl.BlockSpec((16, _D), lambda i: (0, 0)),
            pl.BlockSpec((_NK + 2, _D), lambda i: (0, 0)),
        ],
        out_specs=pl.BlockSpec((_BLK, _D), lambda i: (i, 0)),
        out_shape=jax.ShapeDtypeStruct((_B, _D), jnp.float32),
    )(g3, h3, a2, wcat, t16, coef)


# speedup vs baseline: 14.6237x; 1.1187x over previous
import jax
import jax.numpy as jnp
import numpy as np
from jax import lax
from jax.experimental import pallas as pl

_B = 16384
_D = 128
_HALF = 64
_BLK = 4096
_NBLK = _B // _BLK
_GROUPS = _BLK // 128

_CC = [0.997372368562427, -19.525529325526072, 60.98837617328467,
       -59.53458698148354]
_SC = [6.281264969274094, -41.18603057771831, 78.74175287540852,
       -58.10819811234971]
_NK = 4

_lane_is_cos = np.arange(_D) < _HALF
_COEF = np.stack(
    [np.where(_lane_is_cos, c, s).astype(np.float32)
     for c, s in zip(_CC, _SC)]
    + [_lane_is_cos.astype(np.float32), (~_lane_is_cos).astype(np.float32)]
)


def _fused_body(g_ref, h_ref, age_ref, w_ref, t16_ref, coef_ref, out_ref):
    age = age_ref[...]
    age = jnp.where(jnp.isnan(age), jnp.zeros_like(age), age)
    ageT = jnp.transpose(age)

    g = g_ref[0]
    h = h_ref[0]
    iot = lax.broadcasted_iota(jnp.int32, (8, _BLK), 0)
    oh8 = ((iot == g) | (iot == (h + 3))).astype(jnp.bfloat16)
    lhs_tab = jnp.concatenate([oh8, oh8], axis=0)
    dn = (((0,), (0,)), ((), ()))
    tab = lax.dot_general(lhs_tab, t16_ref[...], dn,
                          preferred_element_type=jnp.float32)

    w = w_ref[...]
    mcos = coef_ref[_NK:_NK + 1, :]
    msin = coef_ref[_NK + 1:_NK + 2, :]
    crows = [coef_ref[k:k + 1, :] for k in range(_NK)]

    for j in range(_GROUPS):
        a = ageT[:, j:j + 1]
        t = a * w
        r = t - jnp.round(t)
        x = r * r
        acc = jnp.broadcast_to(crows[_NK - 1], t.shape)
        for k in range(_NK - 2, -1, -1):
            acc = acc * x + crows[k]
        m = mcos + r * msin
        out_ref[j * 128:(j + 1) * 128, :] = (
            acc * m + tab[j * 128:(j + 1) * 128, :])


@jax.jit
def kernel(gender_labels, health_labels, age_values, gender_table,
           health_table, fourier_weight):
    g3 = gender_labels.astype(jnp.int32).reshape(_NBLK, 1, _BLK)
    h3 = health_labels.astype(jnp.int32).reshape(_NBLK, 1, _BLK)
    a2 = age_values.reshape(_B // _D, _D)

    wrow = fourier_weight.reshape(1, _HALF)
    wcat = jnp.concatenate([wrow, wrow], axis=1)

    gt_hi = gender_table.astype(jnp.bfloat16)
    gt_lo = (gender_table - gt_hi.astype(jnp.float32)).astype(jnp.bfloat16)
    ht_hi = health_table.astype(jnp.bfloat16)
    ht_lo = (health_table - ht_hi.astype(jnp.float32)).astype(jnp.bfloat16)
    z2 = jnp.zeros((2, _D), jnp.bfloat16)
    t16 = jnp.concatenate([gt_hi, ht_hi, z2, gt_lo, ht_lo, z2], axis=0)

    coef = jnp.asarray(_COEF)

    grid = (_NBLK,)
    return pl.pallas_call(
        _fused_body,
        grid=grid,
        in_specs=[
            pl.BlockSpec((1, 1, _BLK), lambda i: (i, 0, 0)),
            pl.BlockSpec((1, 1, _BLK), lambda i: (i, 0, 0)),
            pl.BlockSpec((_GROUPS, _D), lambda i: (i, 0)),
            pl.BlockSpec((1, _D), lambda i: (0, 0)),
            pl.BlockSpec((16, _D), lambda i: (0, 0)),
            pl.BlockSpec((_NK + 2, _D), lambda i: (0, 0)),
        ],
        out_specs=pl.BlockSpec((_BLK, _D), lambda i: (i, 0)),
        out_shape=jax.ShapeDtypeStruct((_B, _D), jnp.float32),
    )(g3, h3, a2, wcat, t16, coef)


# speedup vs baseline: 17.6318x; 1.2057x over previous
import jax
import jax.numpy as jnp
import numpy as np
from jax import lax
from jax.experimental import pallas as pl

_B = 16384
_D = 128
_HALF = 64
_BLK = 4096
_NBLK = _B // _BLK
_GROUPS = _BLK // 128

_CC = [0.997372368562427, -19.525529325526072, 60.98837617328467,
       -59.53458698148354]
_SC = [6.281264969274094, -41.18603057771831, 78.74175287540852,
       -58.10819811234971]
_NK = 4

_lane_is_cos = np.arange(_D) < _HALF
_COEF = np.stack(
    [np.where(_lane_is_cos, c, s).astype(np.float32)
     for c, s in zip(_CC, _SC)]
    + [_lane_is_cos.astype(np.float32), (~_lane_is_cos).astype(np.float32)]
)


def _fused_body(g_ref, h_ref, age_ref, gt_ref, ht_ref, w_ref, coef_ref,
                out_ref):
    age = age_ref[...]
    age = jnp.where(jnp.isnan(age), jnp.zeros_like(age), age)
    ageT = jnp.transpose(age)

    gt = gt_ref[...]
    ht = ht_ref[...]
    gt_hi = gt.astype(jnp.bfloat16)
    ht_hi = ht.astype(jnp.bfloat16)
    gt_lo = (gt - gt_hi.astype(jnp.float32)).astype(jnp.bfloat16)
    ht_lo = (ht - ht_hi.astype(jnp.float32)).astype(jnp.bfloat16)
    z2 = jnp.zeros((2, _D), jnp.bfloat16)
    t16 = jnp.concatenate([gt_hi, ht_hi, z2, gt_lo, ht_lo, z2], axis=0)

    wrow = jnp.transpose(w_ref[...])
    w = jnp.concatenate([wrow, wrow], axis=1)

    mcos = coef_ref[_NK:_NK + 1, :]
    msin = coef_ref[_NK + 1:_NK + 2, :]
    crows = [coef_ref[k:k + 1, :] for k in range(_NK)]
    iot = lax.broadcasted_iota(jnp.int32, (8, _D), 0)
    dn = (((0,), (0,)), ((), ()))

    for j in range(_GROUPS):
        g = g_ref[j:j + 1, :]
        h = h_ref[j:j + 1, :]
        oh8 = ((iot == g) | (iot == (h + 3))).astype(jnp.bfloat16)
        oh16 = jnp.concatenate([oh8, oh8], axis=0)
        tab = lax.dot_general(oh16, t16, dn,
                              preferred_element_type=jnp.float32)

        a = ageT[:, j:j + 1]
        t = a * w
        r = t - jnp.round(t)
        x = r * r
        acc = jnp.broadcast_to(crows[_NK - 1], t.shape)
        for k in range(_NK - 2, -1, -1):
            acc = acc * x + crows[k]
        m = mcos + r * msin
        out_ref[j * 128:(j + 1) * 128, :] = acc * m + tab


@jax.jit
def kernel(gender_labels, health_labels, age_values, gender_table,
           health_table, fourier_weight):
    g2 = gender_labels.astype(jnp.int32).reshape(_B // _D, _D)
    h2 = health_labels.astype(jnp.int32).reshape(_B // _D, _D)
    a2 = age_values.reshape(_B // _D, _D)

    grid = (_NBLK,)
    return pl.pallas_call(
        _fused_body,
        grid=grid,
        in_specs=[
            pl.BlockSpec((_GROUPS, _D), lambda i: (i, 0)),
            pl.BlockSpec((_GROUPS, _D), lambda i: (i, 0)),
            pl.BlockSpec((_GROUPS, _D), lambda i: (i, 0)),
            pl.BlockSpec((3, _D), lambda i: (0, 0)),
            pl.BlockSpec((3, _D), lambda i: (0, 0)),
            pl.BlockSpec((_HALF, 1), lambda i: (0, 0)),
            pl.BlockSpec((_NK + 2, _D), lambda i: (0, 0)),
        ],
        out_specs=pl.BlockSpec((_BLK, _D), lambda i: (i, 0)),
        out_shape=jax.ShapeDtypeStruct((_B, _D), jnp.float32),
    )(g2, h2, a2, gender_table, health_table, fourier_weight,
      jnp.asarray(_COEF))
